# Initial kernel scaffold; baseline (speedup 1.0000x reference)
import jax
import jax.numpy as jnp
from jax.experimental import pallas as pl


def kernel(atomic_embedding, partial_charges, pair_indices, f_ij_cutoff, r_ij, W_emb, b_emb, W_charge, b_charge):
    raise NotImplementedError("write your pallas kernel here")


# speedup vs baseline: 1.3349x; 1.3349x over previous
import dataclasses
import functools

import jax
import jax.numpy as jnp
from jax import lax
from jax.experimental import pallas as pl
from jax.experimental.pallas import tpu as pltpu
from jax.experimental.pallas import tpu_sc as plsc

N_NODES = 10000
N_EDGES = 160000
N_FEAT = 256

NCORE = 2
NSUB = 16
ECORE = N_EDGES // NCORE
SEG = 2000
NSEGS = ECORE // SEG
NVSEG = SEG // 16
NPAD = 10240
NT = NPAD // NSUB
CHT = 160
NCHT = NT // CHT
BLK = 32
EBUF = SEG + BLK + 16


def _unit_body(r_ref, o_ref):
    r = r_ref[...]
    nrm = jnp.sqrt(jnp.sum(r * r, axis=1, keepdims=True))
    u = r / nrm
    o_ref[...] = jnp.concatenate([u, jnp.zeros((r.shape[0], 125), r.dtype)],
                                 axis=1)


def _unit_pad(r):
    B = 8000
    return pl.pallas_call(
        _unit_body,
        grid=(N_EDGES // B,),
        in_specs=[pl.BlockSpec((B, 3), lambda i: (i, 0))],
        out_specs=pl.BlockSpec((B, 128), lambda i: (i, 0)),
        out_shape=jax.ShapeDtypeStruct((N_EDGES, 128), jnp.float32),
    )(r)


def _sc_run(idxj, f, emb, chg, u):
    mesh = plsc.VectorSubcoreMesh(core_axis_name="core",
                                  subcore_axis_name="subcore")
    out_type = [
        jax.ShapeDtypeStruct((NCORE, NPAD, N_FEAT), jnp.float32),
        jax.ShapeDtypeStruct((NCORE, NPAD, N_FEAT), jnp.float32),
        jax.ShapeDtypeStruct((NCORE, NPAD * 32), jnp.float32),
    ]
    scratch = [
        pltpu.VMEM((SEG,), jnp.int32),
        pltpu.VMEM((EBUF,), jnp.int32),
        pltpu.VMEM((EBUF,), jnp.int32),
        pltpu.VMEM((BLK, N_FEAT), jnp.float32),
        pltpu.VMEM((BLK, N_FEAT), jnp.float32),
        pltpu.VMEM((BLK, N_FEAT), jnp.float32),
        pltpu.VMEM((BLK, 128), jnp.float32),
        pltpu.VMEM((CHT, N_FEAT), jnp.float32),
        pltpu.VMEM((CHT, N_FEAT), jnp.float32),
        pltpu.VMEM((CHT * 32,), jnp.float32),
        pltpu.SemaphoreType.DMA,
    ]

    cp = pltpu.CompilerParams()
    if "needs_layout_passes" in pltpu.CompilerParams.__dataclass_fields__:
        cp = dataclasses.replace(cp, needs_layout_passes=False)

    @functools.partial(pl.kernel, out_type=out_type, mesh=mesh,
                       scratch_types=scratch, compiler_params=cp)
    def body(idxj_hbm, f_hbm, emb_hbm, chg_hbm, u_hbm,
             pe_hbm, pc_hbm, pv_hbm,
             segb, eid_v, lv_v, fb, geb, gcb, ub,
             acc_e, acc_c, vacc, sem_g):
        cid = lax.axis_index("core")
        sid = lax.axis_index("subcore")
        ebase = cid * ECORE
        tb = sid * NT
        lane = lax.iota(jnp.int32, 16)
        zvec = jnp.zeros((16,), jnp.float32)

        @pl.loop(0, NCHT)
        def _chunk(ch):
            cb = tb + ch * CHT

            @pl.loop(0, CHT)
            def _zr(r):
                @pl.loop(0, N_FEAT, step=16)
                def _zc(k):
                    acc_e[r, pl.ds(k, 16)] = zvec
                    acc_c[r, pl.ds(k, 16)] = zvec
                @pl.loop(0, 32, step=16)
                def _zc2(k):
                    vacc[pl.ds(r * 32 + k, 16)] = zvec

            embv = emb_hbm.at[pl.ds(cb, CHT)]
            chgv = chg_hbm.at[pl.ds(cb, CHT)]

            @pl.loop(0, NSEGS)
            def _seg(g):
                pltpu.sync_copy(idxj_hbm.at[pl.ds(ebase + g * SEG, SEG)], segb)

                def comp(v, off):
                    vec = segb[pl.ds(v * 16, 16)]
                    m = (vec >= cb) & (vec < cb + CHT)
                    plsc.store_compressed(lv_v.at[pl.ds(off, 16)], vec - cb,
                                          mask=m)
                    ev = ebase + g * SEG + v * 16 + lane
                    plsc.store_compressed(eid_v.at[pl.ds(off, 16)], ev,
                                          mask=m)
                    return off + jnp.sum(m.astype(jnp.int32))

                n = lax.fori_loop(0, NVSEG, comp, jnp.int32(0))

                @pl.loop(0, BLK // 16)
                def _pad(t):
                    eid_v[pl.ds(n + t * 16, 16)] = jnp.broadcast_to(
                        jnp.int32(0) + ebase, (16,))
                    lv_v[pl.ds(n + t * 16, 16)] = jnp.broadcast_to(
                        jnp.int32(0), (16,))

                nb = (n + (BLK - 1)) // BLK

                def blk(b, carry):
                    o = b * BLK
                    cps = [
                        pltpu.async_copy(f_hbm.at[eid_v.at[pl.ds(o, BLK)]],
                                         fb, sem_g),
                        pltpu.async_copy(u_hbm.at[eid_v.at[pl.ds(o, BLK)]],
                                         ub, sem_g),
                        pltpu.async_copy(embv.at[lv_v.at[pl.ds(o, BLK)]],
                                         geb, sem_g),
                        pltpu.async_copy(chgv.at[lv_v.at[pl.ds(o, BLK)]],
                                         gcb, sem_g),
                    ]
                    for cpd in cps:
                        cpd.wait()

                    def edge(e, carry2):
                        lv = lv_v[pl.ds(o + e, 16)][0]
                        valid = (o + e) < n
                        se = zvec
                        sc = zvec
                        for k in range(N_FEAT // 16):
                            fv = jnp.where(valid, fb[e, pl.ds(k * 16, 16)],
                                           zvec)
                            pe = fv * geb[e, pl.ds(k * 16, 16)]
                            pc = fv * gcb[e, pl.ds(k * 16, 16)]
                            plsc.addupdate(acc_e.at[lv, pl.ds(k * 16, 16)], pe)
                            plsc.addupdate(acc_c.at[lv, pl.ds(k * 16, 16)], pc)
                            se = se + pe
                            sc = sc + pc
                        uvec = ub[e, pl.ds(0, 16)]
                        plsc.addupdate(vacc.at[pl.ds(lv * 32, 16)],
                                       uvec * jnp.sum(se))
                        plsc.addupdate(vacc.at[pl.ds(lv * 32 + 16, 16)],
                                       uvec * jnp.sum(sc))
                        return carry2

                    lax.fori_loop(0, BLK, edge, 0)
                    return carry

                lax.fori_loop(0, nb, blk, 0)

            pltpu.sync_copy(acc_e, pe_hbm.at[cid, pl.ds(cb, CHT)])
            pltpu.sync_copy(acc_c, pc_hbm.at[cid, pl.ds(cb, CHT)])
            pltpu.sync_copy(vacc, pv_hbm.at[cid, pl.ds(cb * 32, CHT * 32)])

    return body(idxj, f, emb, chg, u)


def _post_body(pe_ref, pc_ref, pv_ref, re_ref, ve_ref, rc_ref, vc_ref):
    re_ref[...] = pe_ref[0] + pe_ref[1]
    rc_ref[...] = pc_ref[0] + pc_ref[1]
    v = pv_ref[0] + pv_ref[1]
    ve = v[:, 0:16]
    vc = v[:, 16:32]
    ve_ref[...] = jnp.sqrt(jnp.sum(ve * ve, axis=1, keepdims=True))
    vc_ref[...] = jnp.sqrt(jnp.sum(vc * vc, axis=1, keepdims=True))


def _post(pe, pc, pv):
    B = 2000
    return pl.pallas_call(
        _post_body,
        grid=(N_NODES // B,),
        in_specs=[
            pl.BlockSpec((NCORE, B, N_FEAT), lambda i: (0, i, 0)),
            pl.BlockSpec((NCORE, B, N_FEAT), lambda i: (0, i, 0)),
            pl.BlockSpec((NCORE, B, 32), lambda i: (0, i, 0)),
        ],
        out_specs=[
            pl.BlockSpec((B, N_FEAT), lambda i: (i, 0)),
            pl.BlockSpec((B, 1), lambda i: (i, 0)),
            pl.BlockSpec((B, N_FEAT), lambda i: (i, 0)),
            pl.BlockSpec((B, 1), lambda i: (i, 0)),
        ],
        out_shape=[
            jax.ShapeDtypeStruct((N_NODES, N_FEAT), jnp.float32),
            jax.ShapeDtypeStruct((N_NODES, 1), jnp.float32),
            jax.ShapeDtypeStruct((N_NODES, N_FEAT), jnp.float32),
            jax.ShapeDtypeStruct((N_NODES, 1), jnp.float32),
        ],
    )(pe, pc, pv)


def kernel(atomic_embedding, partial_charges, pair_indices, f_ij_cutoff, r_ij,
           W_emb, b_emb, W_charge, b_charge):
    idxj = pair_indices[1]
    u = _unit_pad(r_ij)
    emb_p = jnp.pad(atomic_embedding, ((0, NPAD - N_NODES), (0, 0)))
    chg_p = jnp.pad(partial_charges, ((0, NPAD - N_NODES), (0, 0)))
    pe, pc, pv = _sc_run(idxj, f_ij_cutoff, emb_p, chg_p, u)
    pv = pv.reshape(NCORE, NPAD, 32)
    rad_e, vn_e, rad_c, vn_c = _post(pe, pc, pv)
    return jnp.concatenate([rad_e, vn_e, rad_c, vn_c], axis=1)
